# baseline (speedup 1.0000x reference)
import functools

import jax
import jax.numpy as jnp
from jax import lax
from jax.experimental import pallas as pl
from jax.experimental.pallas import tpu as pltpu
from jax.experimental.pallas import tpu_sc as plsc

_C = 21
_B = 8
_PIX = 512 * 512
_NW = 32
_PER_W = _B * _PIX // _NW
_CHUNK = 16384
_NCH = _PER_W // _CHUNK
_LANES = 16
_NB = 4
_HBINS = _C * _LANES
_HSIZE = _NB * _HBINS
_UNROLL = 8


def _sc_hist_body(pred_hbm, label_hbm, out_hbm, pbuf, lbuf, hp, hl):
    wid = lax.axis_index("s") * 2 + lax.axis_index("c")
    base = wid * _PER_W

    zeros = jnp.zeros((_LANES,), jnp.int32)
    for c in range(_HSIZE // _LANES):
        hp[pl.ds(c * _LANES, _LANES)] = zeros
        hl[pl.ds(c * _LANES, _LANES)] = zeros

    lane = lax.iota(jnp.int32, _LANES)
    ones = jnp.ones((_LANES,), jnp.int32)

    for ch in range(_NCH):
        off = base + ch * _CHUNK
        pltpu.sync_copy(pred_hbm.at[pl.ds(off, _CHUNK)], pbuf)
        pltpu.sync_copy(label_hbm.at[pl.ds(off, _CHUNK)], lbuf)

        def body(i, carry):
            for u in range(_UNROLL):
                s = pl.ds((i * _UNROLL + u) * _LANES, _LANES)
                p = pbuf[s]
                l = lbuf[s]
                boff = (u % _NB) * _HBINS
                pi = lax.shift_left(p, 4) + (lane + boff)
                li = lax.shift_left(l, 4) + (lane + boff)
                v = jnp.where(p == l, jnp.int32(8193), jnp.int32(1))
                plsc.addupdate_scatter(hp, [pi], v)
                plsc.addupdate_scatter(hl, [li], ones)
            return carry

        lax.fori_loop(0, _CHUNK // (_UNROLL * _LANES), body, 0)

    obase = wid * 2 * _HSIZE
    pltpu.sync_copy(hp, out_hbm.at[pl.ds(obase, _HSIZE)])
    pltpu.sync_copy(hl, out_hbm.at[pl.ds(obase + _HSIZE, _HSIZE)])


@functools.cache
def _sc_hist():
    return pl.kernel(
        _sc_hist_body,
        out_type=jax.ShapeDtypeStruct((_NW * 2 * _HSIZE,), jnp.int32),
        mesh=plsc.VectorSubcoreMesh(core_axis_name="c", subcore_axis_name="s"),
        compiler_params=pltpu.CompilerParams(needs_layout_passes=False),
        scratch_types=[
            pltpu.VMEM((_CHUNK,), jnp.int32),
            pltpu.VMEM((_CHUNK,), jnp.int32),
            pltpu.VMEM((_HSIZE,), jnp.int32),
            pltpu.VMEM((_HSIZE,), jnp.int32),
        ],
    )


_ROW = 4 * 2 * _HSIZE


def _combine_body(parts_ref, out_ref):
    x = parts_ref[...]
    lo = jnp.bitwise_and(x, 8191).astype(jnp.float32)
    hi = lax.shift_right_logical(x, 13).astype(jnp.float32)
    k = lax.broadcasted_iota(jnp.int32, (_ROW, 2 * _C), 0)
    m = lax.broadcasted_iota(jnp.int32, (_ROW, 2 * _C), 1)
    sel = ((k % (2 * _HSIZE)) // _HSIZE == m // _C) & (
        (k % _HBINS) // _LANES == m % _C
    )
    s = sel.astype(jnp.float32)
    ylo = jnp.dot(lo, s, preferred_element_type=jnp.float32)
    yhi = jnp.dot(hi, s, preferred_element_type=jnp.float32)
    cp = ylo[:, 0:_C]
    cl = ylo[:, _C:2 * _C]
    cb = yhi[:, 0:_C]
    dice = (2.0 * cb) / (cp + cl + 1e-10)
    out_ref[...] = jnp.mean(dice, axis=0, keepdims=True)


def kernel(pred, label):
    pred_flat = pred.reshape(_B * _PIX)
    label_flat = label.reshape(_B * _PIX)
    parts = _sc_hist()(pred_flat, label_flat)
    parts2 = parts.reshape(_B, _ROW)
    out = pl.pallas_call(
        _combine_body,
        out_shape=jax.ShapeDtypeStruct((1, _C), jnp.float32),
    )(parts2)
    return out.reshape(_C)


# speedup vs baseline: 1.2746x; 1.2746x over previous
import functools

import jax
import jax.numpy as jnp
from jax import lax
from jax.experimental import pallas as pl
from jax.experimental.pallas import tpu as pltpu
from jax.experimental.pallas import tpu_sc as plsc

_C = 21
_B = 8
_PIX = 512 * 512
_NW = 32
_PER_W = _B * _PIX // _NW
_CHUNK = 16384
_NCH = _PER_W // _CHUNK
_LANES = 16
_NB = 4
_HBINS = _C * _LANES
_HSIZE = _NB * _HBINS
_UNROLL = 8


def _sc_hist_body(pred_hbm, label_hbm, out_hbm, pbuf, lbuf, hp, hl):
    wid = lax.axis_index("s") * 2 + lax.axis_index("c")
    base = wid * _PER_W

    zeros = jnp.zeros((_LANES,), jnp.int32)
    for c in range(_HSIZE // _LANES):
        hp[pl.ds(c * _LANES, _LANES)] = zeros
        hl[pl.ds(c * _LANES, _LANES)] = zeros

    lane = lax.iota(jnp.int32, _LANES)
    ones = jnp.ones((_LANES,), jnp.int32)

    for ch in range(_NCH):
        off = base + ch * _CHUNK
        pltpu.sync_copy(pred_hbm.at[pl.ds(off, _CHUNK)], pbuf)
        pltpu.sync_copy(label_hbm.at[pl.ds(off, _CHUNK)], lbuf)

        @plsc.parallel_loop(0, _CHUNK // _LANES, unroll=_UNROLL)
        def body(i):
            s = pl.ds(i * _LANES, _LANES)
            p = pbuf[s]
            l = lbuf[s]
            boff = (i & (_NB - 1)) * _HBINS
            pi = lax.shift_left(p, 4) + (lane + boff)
            li = lax.shift_left(l, 4) + (lane + boff)
            v = jnp.where(p == l, jnp.int32(8193), jnp.int32(1))
            plsc.addupdate_scatter(hp, [pi], v)
            plsc.addupdate_scatter(hl, [li], ones)

    obase = wid * 2 * _HSIZE
    pltpu.sync_copy(hp, out_hbm.at[pl.ds(obase, _HSIZE)])
    pltpu.sync_copy(hl, out_hbm.at[pl.ds(obase + _HSIZE, _HSIZE)])


@functools.cache
def _sc_hist():
    return pl.kernel(
        _sc_hist_body,
        out_type=jax.ShapeDtypeStruct((_NW * 2 * _HSIZE,), jnp.int32),
        mesh=plsc.VectorSubcoreMesh(core_axis_name="c", subcore_axis_name="s"),
        compiler_params=pltpu.CompilerParams(needs_layout_passes=False),
        scratch_types=[
            pltpu.VMEM((_CHUNK,), jnp.int32),
            pltpu.VMEM((_CHUNK,), jnp.int32),
            pltpu.VMEM((_HSIZE,), jnp.int32),
            pltpu.VMEM((_HSIZE,), jnp.int32),
        ],
    )


_ROW = 4 * 2 * _HSIZE


def _combine_body(parts_ref, out_ref):
    x = parts_ref[...]
    lo = jnp.bitwise_and(x, 8191).astype(jnp.float32)
    hi = lax.shift_right_logical(x, 13).astype(jnp.float32)
    k = lax.broadcasted_iota(jnp.int32, (_ROW, 2 * _C), 0)
    m = lax.broadcasted_iota(jnp.int32, (_ROW, 2 * _C), 1)
    sel = ((k % (2 * _HSIZE)) // _HSIZE == m // _C) & (
        (k % _HBINS) // _LANES == m % _C
    )
    s = sel.astype(jnp.float32)
    ylo = jnp.dot(lo, s, preferred_element_type=jnp.float32)
    yhi = jnp.dot(hi, s, preferred_element_type=jnp.float32)
    cp = ylo[:, 0:_C]
    cl = ylo[:, _C:2 * _C]
    cb = yhi[:, 0:_C]
    dice = (2.0 * cb) / (cp + cl + 1e-10)
    out_ref[...] = jnp.mean(dice, axis=0, keepdims=True)


def kernel(pred, label):
    pred_flat = pred.reshape(_B * _PIX)
    label_flat = label.reshape(_B * _PIX)
    parts = _sc_hist()(pred_flat, label_flat)
    parts2 = parts.reshape(_B, _ROW)
    out = pl.pallas_call(
        _combine_body,
        out_shape=jax.ShapeDtypeStruct((1, _C), jnp.float32),
    )(parts2)
    return out.reshape(_C)


# speedup vs baseline: 1.4108x; 1.1068x over previous
import functools

import jax
import jax.numpy as jnp
from jax import lax
from jax.experimental import pallas as pl
from jax.experimental.pallas import tpu as pltpu
from jax.experimental.pallas import tpu_sc as plsc

_C = 21
_B = 8
_PIX = 512 * 512
_NW = 32
_PER_W = _B * _PIX // _NW
_CHUNK = 16384
_NCH = _PER_W // _CHUNK
_LANES = 16
_NB = 4
_HBINS = _C * _LANES
_HSIZE = _NB * _HBINS
_UNROLL = 8


def _sc_hist_body(
    pred_hbm, label_hbm, out_hbm, pa, pb, la, lb, hp, hl, s0, s1, s2, s3
):
    wid = lax.axis_index("s") * 2 + lax.axis_index("c")
    base = wid * _PER_W

    pbufs = (pa, pb)
    lbufs = (la, lb)
    sems = ((s0, s1), (s2, s3))

    def start(ch):
        slot = ch % 2
        off = base + ch * _CHUNK
        return (
            pltpu.async_copy(
                pred_hbm.at[pl.ds(off, _CHUNK)], pbufs[slot], sems[slot][0]
            ),
            pltpu.async_copy(
                label_hbm.at[pl.ds(off, _CHUNK)], lbufs[slot], sems[slot][1]
            ),
        )

    pending = start(0)

    zeros = jnp.zeros((_LANES,), jnp.int32)
    for c in range(_HSIZE // _LANES):
        hp[pl.ds(c * _LANES, _LANES)] = zeros
        hl[pl.ds(c * _LANES, _LANES)] = zeros

    lane = lax.iota(jnp.int32, _LANES)
    ones = jnp.ones((_LANES,), jnp.int32)

    for ch in range(_NCH):
        dp, dl = pending
        if ch + 1 < _NCH:
            pending = start(ch + 1)
        dp.wait()
        dl.wait()
        pbuf = pbufs[ch % 2]
        lbuf = lbufs[ch % 2]

        @plsc.parallel_loop(0, _CHUNK // _LANES, unroll=_UNROLL)
        def body(i):
            s = pl.ds(i * _LANES, _LANES)
            p = pbuf[s]
            l = lbuf[s]
            boff = (i & (_NB - 1)) * _HBINS
            pi = lax.shift_left(p, 4) + (lane + boff)
            li = lax.shift_left(l, 4) + (lane + boff)
            v = jnp.where(p == l, jnp.int32(8193), jnp.int32(1))
            plsc.addupdate_scatter(hp, [pi], v)
            plsc.addupdate_scatter(hl, [li], ones)

    obase = wid * 2 * _HSIZE
    pltpu.sync_copy(hp, out_hbm.at[pl.ds(obase, _HSIZE)])
    pltpu.sync_copy(hl, out_hbm.at[pl.ds(obase + _HSIZE, _HSIZE)])


@functools.cache
def _sc_hist():
    return pl.kernel(
        _sc_hist_body,
        out_type=jax.ShapeDtypeStruct((_NW * 2 * _HSIZE,), jnp.int32),
        mesh=plsc.VectorSubcoreMesh(core_axis_name="c", subcore_axis_name="s"),
        compiler_params=pltpu.CompilerParams(needs_layout_passes=False),
        scratch_types=[
            pltpu.VMEM((_CHUNK,), jnp.int32),
            pltpu.VMEM((_CHUNK,), jnp.int32),
            pltpu.VMEM((_CHUNK,), jnp.int32),
            pltpu.VMEM((_CHUNK,), jnp.int32),
            pltpu.VMEM((_HSIZE,), jnp.int32),
            pltpu.VMEM((_HSIZE,), jnp.int32),
            pltpu.SemaphoreType.DMA,
            pltpu.SemaphoreType.DMA,
            pltpu.SemaphoreType.DMA,
            pltpu.SemaphoreType.DMA,
        ],
    )


_ROW = 4 * 2 * _HSIZE


def _combine_body(parts_ref, out_ref):
    x = parts_ref[...]
    lo = jnp.bitwise_and(x, 8191).astype(jnp.float32)
    hi = lax.shift_right_logical(x, 13).astype(jnp.float32)
    k = lax.broadcasted_iota(jnp.int32, (_ROW, 2 * _C), 0)
    m = lax.broadcasted_iota(jnp.int32, (_ROW, 2 * _C), 1)
    sel = ((k % (2 * _HSIZE)) // _HSIZE == m // _C) & (
        (k % _HBINS) // _LANES == m % _C
    )
    s = sel.astype(jnp.float32)
    ylo = jnp.dot(lo, s, preferred_element_type=jnp.float32)
    yhi = jnp.dot(hi, s, preferred_element_type=jnp.float32)
    cp = ylo[:, 0:_C]
    cl = ylo[:, _C:2 * _C]
    cb = yhi[:, 0:_C]
    dice = (2.0 * cb) / (cp + cl + 1e-10)
    out_ref[...] = jnp.mean(dice, axis=0, keepdims=True)


def kernel(pred, label):
    pred_flat = pred.reshape(_B * _PIX)
    label_flat = label.reshape(_B * _PIX)
    parts = _sc_hist()(pred_flat, label_flat)
    parts2 = parts.reshape(_B, _ROW)
    out = pl.pallas_call(
        _combine_body,
        out_shape=jax.ShapeDtypeStruct((1, _C), jnp.float32),
    )(parts2)
    return out.reshape(_C)


# speedup vs baseline: 1.7130x; 1.2142x over previous
import functools

import jax
import jax.numpy as jnp
from jax import lax
from jax.experimental import pallas as pl
from jax.experimental.pallas import tpu as pltpu
from jax.experimental.pallas import tpu_sc as plsc

_C = 21
_B = 8
_PIX = 512 * 512
_W = 512
_ROWS = _B * _PIX // _W
_NW = 32
_WROWS = _ROWS // _NW
_CROWS = 32
_NCH = _WROWS // _CROWS
_LANES = 16
_VPR = _W // _LANES
_NB = 4
_HBINS = _C * _LANES
_HSIZE = _NB * _HBINS


def _sc_hist_body(
    pred_hbm, label_hbm, out_hbm, pa, pb, la, lb, hp, hl, s0, s1, s2, s3
):
    wid = lax.axis_index("s") * 2 + lax.axis_index("c")
    base = wid * _WROWS

    pbufs = (pa, pb)
    lbufs = (la, lb)
    sems = ((s0, s1), (s2, s3))

    def start(ch):
        slot = ch % 2
        r0 = base + ch * _CROWS
        return (
            pltpu.async_copy(
                pred_hbm.at[pl.ds(r0, _CROWS), :], pbufs[slot], sems[slot][0]
            ),
            pltpu.async_copy(
                label_hbm.at[pl.ds(r0, _CROWS), :], lbufs[slot], sems[slot][1]
            ),
        )

    pending = start(0)

    zeros = jnp.zeros((_LANES,), jnp.int32)
    for c in range(_HSIZE // _LANES):
        hp[pl.ds(c * _LANES, _LANES)] = zeros
        hl[pl.ds(c * _LANES, _LANES)] = zeros

    lane = lax.iota(jnp.int32, _LANES)
    ones = jnp.ones((_LANES,), jnp.int32)

    for ch in range(_NCH):
        dp, dl = pending
        if ch + 1 < _NCH:
            pending = start(ch + 1)
        dp.wait()
        dl.wait()
        pbuf = pbufs[ch % 2]
        lbuf = lbufs[ch % 2]

        @plsc.parallel_loop(0, _CROWS, unroll=2)
        def body(r):
            for cv in range(_VPR):
                s = pl.ds(cv * _LANES, _LANES)
                p = pbuf[r, s]
                l = lbuf[r, s]
                boff = (cv % _NB) * _HBINS
                pi = lax.shift_left(p, 4) + (lane + boff)
                li = lax.shift_left(l, 4) + (lane + boff)
                v = jnp.where(p == l, jnp.int32(8193), jnp.int32(1))
                plsc.addupdate_scatter(hp, [pi], v)
                plsc.addupdate_scatter(hl, [li], ones)

    obase = wid * 2 * _HSIZE
    pltpu.sync_copy(hp, out_hbm.at[pl.ds(obase, _HSIZE)])
    pltpu.sync_copy(hl, out_hbm.at[pl.ds(obase + _HSIZE, _HSIZE)])


@functools.cache
def _sc_hist():
    return pl.kernel(
        _sc_hist_body,
        out_type=jax.ShapeDtypeStruct((_NW * 2 * _HSIZE,), jnp.int32),
        mesh=plsc.VectorSubcoreMesh(core_axis_name="c", subcore_axis_name="s"),
        compiler_params=pltpu.CompilerParams(needs_layout_passes=False),
        scratch_types=[
            pltpu.VMEM((_CROWS, _W), jnp.int32),
            pltpu.VMEM((_CROWS, _W), jnp.int32),
            pltpu.VMEM((_CROWS, _W), jnp.int32),
            pltpu.VMEM((_CROWS, _W), jnp.int32),
            pltpu.VMEM((_HSIZE,), jnp.int32),
            pltpu.VMEM((_HSIZE,), jnp.int32),
            pltpu.SemaphoreType.DMA,
            pltpu.SemaphoreType.DMA,
            pltpu.SemaphoreType.DMA,
            pltpu.SemaphoreType.DMA,
        ],
    )


_ROW = 4 * 2 * _HSIZE


def _combine_body(parts_ref, out_ref):
    x = parts_ref[...]
    lo = jnp.bitwise_and(x, 8191).astype(jnp.float32)
    hi = lax.shift_right_logical(x, 13).astype(jnp.float32)
    k = lax.broadcasted_iota(jnp.int32, (_ROW, 2 * _C), 0)
    m = lax.broadcasted_iota(jnp.int32, (_ROW, 2 * _C), 1)
    sel = ((k % (2 * _HSIZE)) // _HSIZE == m // _C) & (
        (k % _HBINS) // _LANES == m % _C
    )
    s = sel.astype(jnp.float32)
    ylo = jnp.dot(lo, s, preferred_element_type=jnp.float32)
    yhi = jnp.dot(hi, s, preferred_element_type=jnp.float32)
    cp = ylo[:, 0:_C]
    cl = ylo[:, _C:2 * _C]
    cb = yhi[:, 0:_C]
    dice = (2.0 * cb) / (cp + cl + 1e-10)
    out_ref[...] = jnp.mean(dice, axis=0, keepdims=True)


def kernel(pred, label):
    pred_rows = pred.reshape(_ROWS, _W)
    label_rows = label.reshape(_ROWS, _W)
    parts = _sc_hist()(pred_rows, label_rows)
    parts2 = parts.reshape(_B, _ROW)
    out = pl.pallas_call(
        _combine_body,
        out_shape=jax.ShapeDtypeStruct((1, _C), jnp.float32),
    )(parts2)
    return out.reshape(_C)


# speedup vs baseline: 2.1025x; 1.2274x over previous
import functools

import jax
import jax.numpy as jnp
from jax import lax
from jax.experimental import pallas as pl
from jax.experimental.pallas import tpu as pltpu
from jax.experimental.pallas import tpu_sc as plsc

_C = 21
_B = 8
_PIX = 512 * 512
_W = 512
_ROWS = _B * _PIX // _W
_NW = 32
_WROWS = _ROWS // _NW
_CROWS = 32
_NCH = _WROWS // _CROWS
_LANES = 16
_VPR = _W // _LANES
_NB = 4
_HBINS = _C * _LANES
_HSIZE = _NB * _HBINS


def _sc_hist_body(
    pred_hbm, label_hbm, out_hbm,
    pa, pb, la, lb, hp, hl, tbuf, obuf, s0, s1, s2, s3,
):
    wid = lax.axis_index("s") * 2 + lax.axis_index("c")
    base = wid * _WROWS

    pbufs = (pa, pb)
    lbufs = (la, lb)
    sems = ((s0, s1), (s2, s3))

    _H = _CROWS // 2

    def start(ch):
        slot = ch % 2
        r0 = base + ch * _CROWS
        return (
            pltpu.async_copy(
                pred_hbm.at[pl.ds(r0, _H), :],
                pbufs[slot].at[pl.ds(0, _H), :], sems[slot][0]
            ),
            pltpu.async_copy(
                pred_hbm.at[pl.ds(r0 + _H, _H), :],
                pbufs[slot].at[pl.ds(_H, _H), :], sems[slot][0]
            ),
            pltpu.async_copy(
                label_hbm.at[pl.ds(r0, _H), :],
                lbufs[slot].at[pl.ds(0, _H), :], sems[slot][1]
            ),
            pltpu.async_copy(
                label_hbm.at[pl.ds(r0 + _H, _H), :],
                lbufs[slot].at[pl.ds(_H, _H), :], sems[slot][1]
            ),
        )

    pending = start(0)

    zeros = jnp.zeros((_LANES,), jnp.int32)
    for c in range(_HSIZE // _LANES):
        hp[pl.ds(c * _LANES, _LANES)] = zeros
        hl[pl.ds(c * _LANES, _LANES)] = zeros

    lane = lax.iota(jnp.int32, _LANES)
    ones = jnp.ones((_LANES,), jnp.int32)

    for ch in range(_NCH):
        descs = pending
        if ch + 1 < _NCH:
            pending = start(ch + 1)
        for d in descs:
            d.wait()
        pbuf = pbufs[ch % 2]
        lbuf = lbufs[ch % 2]

        @plsc.parallel_loop(0, _CROWS, unroll=4)
        def body(r):
            for cv in range(_VPR):
                s = pl.ds(cv * _LANES, _LANES)
                p = pbuf[r, s]
                l = lbuf[r, s]
                boff = (cv % _NB) * _HBINS
                pi = lax.shift_left(p, 4) + (lane + boff)
                li = lax.shift_left(l, 4) + (lane + boff)
                v = jnp.where(p == l, jnp.int32(8193), jnp.int32(1))
                plsc.addupdate_scatter(hp, [pi], v)
                plsc.addupdate_scatter(hl, [li], ones)

    lane64 = lax.shift_left(lane, 6)
    for j in range(_C):
        hpk = (
            hp[pl.ds(j * _LANES, _LANES)]
            + hp[pl.ds(_HBINS + j * _LANES, _LANES)]
            + hp[pl.ds(2 * _HBINS + j * _LANES, _LANES)]
            + hp[pl.ds(3 * _HBINS + j * _LANES, _LANES)]
        )
        lpk = (
            hl[pl.ds(j * _LANES, _LANES)]
            + hl[pl.ds(_HBINS + j * _LANES, _LANES)]
            + hl[pl.ds(2 * _HBINS + j * _LANES, _LANES)]
            + hl[pl.ds(3 * _HBINS + j * _LANES, _LANES)]
        )
        plsc.store_scatter(tbuf, [lane64 + j], jnp.bitwise_and(hpk, 8191))
        plsc.store_scatter(tbuf, [lane64 + (_C + j)], lpk)
        plsc.store_scatter(
            tbuf, [lane64 + (2 * _C + j)], lax.shift_right_logical(hpk, 13)
        )

    accs = [tbuf[pl.ds(v * _LANES, _LANES)] for v in range(4)]
    for blk in range(1, _LANES):
        for v in range(4):
            accs[v] = accs[v] + tbuf[pl.ds(blk * 64 + v * _LANES, _LANES)]
    for v in range(4):
        obuf[pl.ds(v * _LANES, _LANES)] = accs[v]

    pltpu.sync_copy(obuf, out_hbm.at[pl.ds(wid * 64, 64)])


@functools.cache
def _sc_hist():
    return pl.kernel(
        _sc_hist_body,
        out_type=jax.ShapeDtypeStruct((_NW * 64,), jnp.int32),
        mesh=plsc.VectorSubcoreMesh(core_axis_name="c", subcore_axis_name="s"),
        compiler_params=pltpu.CompilerParams(needs_layout_passes=False),
        scratch_types=[
            pltpu.VMEM((_CROWS, _W), jnp.int32),
            pltpu.VMEM((_CROWS, _W), jnp.int32),
            pltpu.VMEM((_CROWS, _W), jnp.int32),
            pltpu.VMEM((_CROWS, _W), jnp.int32),
            pltpu.VMEM((_HSIZE,), jnp.int32),
            pltpu.VMEM((_HSIZE,), jnp.int32),
            pltpu.VMEM((_LANES * 64,), jnp.int32),
            pltpu.VMEM((64,), jnp.int32),
            pltpu.SemaphoreType.DMA,
            pltpu.SemaphoreType.DMA,
            pltpu.SemaphoreType.DMA,
            pltpu.SemaphoreType.DMA,
        ],
    )


def _combine_body(parts_ref, out_ref):
    x = parts_ref[...].astype(jnp.float32)
    cp = jnp.zeros((_B, _C), jnp.float32)
    cl = jnp.zeros((_B, _C), jnp.float32)
    cb = jnp.zeros((_B, _C), jnp.float32)
    for q in range(4):
        o = q * 64
        cp = cp + x[:, o:o + _C]
        cl = cl + x[:, o + _C:o + 2 * _C]
        cb = cb + x[:, o + 2 * _C:o + 3 * _C]
    dice = (2.0 * cb) / (cp + cl + 1e-10)
    out_ref[...] = jnp.mean(dice, axis=0, keepdims=True)


def kernel(pred, label):
    pred_rows = pred.reshape(_ROWS, _W)
    label_rows = label.reshape(_ROWS, _W)
    parts = _sc_hist()(pred_rows, label_rows)
    parts2 = parts.reshape(_B, 4 * 64)
    out = pl.pallas_call(
        _combine_body,
        out_shape=jax.ShapeDtypeStruct((1, _C), jnp.float32),
    )(parts2)
    return out.reshape(_C)


# speedup vs baseline: 2.1101x; 1.0036x over previous
import functools

import jax
import jax.numpy as jnp
from jax import lax
from jax.experimental import pallas as pl
from jax.experimental.pallas import tpu as pltpu
from jax.experimental.pallas import tpu_sc as plsc

_C = 21
_B = 8
_PIX = 512 * 512
_W = 512
_ROWS = _B * _PIX // _W
_NW = 32
_WROWS = _ROWS // _NW
_CROWS = 32
_NCH = _WROWS // _CROWS
_LANES = 16
_VPR = _W // _LANES
_NB = 4
_HBINS = _C * _LANES
_HSIZE = _NB * _HBINS


def _sc_hist_body(
    pred_hbm, label_hbm, out_hbm,
    pa, pb, la, lb, hp, hl, tbuf, obuf, s0, s1, s2, s3,
):
    wid = lax.axis_index("s") * 2 + lax.axis_index("c")
    base = wid * _WROWS

    pbufs = (pa, pb)
    lbufs = (la, lb)
    sems = ((s0, s1), (s2, s3))

    def start(ch):
        slot = ch % 2
        r0 = base + ch * _CROWS
        return (
            pltpu.async_copy(
                pred_hbm.at[pl.ds(r0, _CROWS), :], pbufs[slot], sems[slot][0]
            ),
            pltpu.async_copy(
                label_hbm.at[pl.ds(r0, _CROWS), :], lbufs[slot], sems[slot][1]
            ),
        )

    pending = start(0)

    zeros = jnp.zeros((_LANES,), jnp.int32)
    for c in range(_HSIZE // _LANES):
        hp[pl.ds(c * _LANES, _LANES)] = zeros
        hl[pl.ds(c * _LANES, _LANES)] = zeros

    lane = lax.iota(jnp.int32, _LANES)
    ones = jnp.ones((_LANES,), jnp.int32)

    for ch in range(_NCH):
        dp, dl = pending
        if ch + 1 < _NCH:
            pending = start(ch + 1)
        dp.wait()
        dl.wait()
        pbuf = pbufs[ch % 2]
        lbuf = lbufs[ch % 2]

        @plsc.parallel_loop(0, _CROWS, unroll=4)
        def body(r):
            for cv in range(_VPR):
                s = pl.ds(cv * _LANES, _LANES)
                p = pbuf[r, s]
                l = lbuf[r, s]
                boff = (cv % _NB) * _HBINS
                pi = lax.shift_left(p, 4) + (lane + boff)
                li = lax.shift_left(l, 4) + (lane + boff)
                v = jnp.where(p == l, jnp.int32(8193), jnp.int32(1))
                plsc.addupdate_scatter(hp, [pi], v)
                plsc.addupdate_scatter(hl, [li], ones)

    lane64 = lax.shift_left(lane, 6)
    for j in range(_C):
        hpk = (
            hp[pl.ds(j * _LANES, _LANES)]
            + hp[pl.ds(_HBINS + j * _LANES, _LANES)]
            + hp[pl.ds(2 * _HBINS + j * _LANES, _LANES)]
            + hp[pl.ds(3 * _HBINS + j * _LANES, _LANES)]
        )
        lpk = (
            hl[pl.ds(j * _LANES, _LANES)]
            + hl[pl.ds(_HBINS + j * _LANES, _LANES)]
            + hl[pl.ds(2 * _HBINS + j * _LANES, _LANES)]
            + hl[pl.ds(3 * _HBINS + j * _LANES, _LANES)]
        )
        plsc.store_scatter(tbuf, [lane64 + j], jnp.bitwise_and(hpk, 8191))
        plsc.store_scatter(tbuf, [lane64 + (_C + j)], lpk)
        plsc.store_scatter(
            tbuf, [lane64 + (2 * _C + j)], lax.shift_right_logical(hpk, 13)
        )

    accs = [tbuf[pl.ds(v * _LANES, _LANES)] for v in range(4)]
    for blk in range(1, _LANES):
        for v in range(4):
            accs[v] = accs[v] + tbuf[pl.ds(blk * 64 + v * _LANES, _LANES)]
    for v in range(4):
        obuf[pl.ds(v * _LANES, _LANES)] = accs[v]

    pltpu.sync_copy(obuf, out_hbm.at[pl.ds(wid * 64, 64)])


@functools.cache
def _sc_hist():
    return pl.kernel(
        _sc_hist_body,
        out_type=jax.ShapeDtypeStruct((_NW * 64,), jnp.int32),
        mesh=plsc.VectorSubcoreMesh(core_axis_name="c", subcore_axis_name="s"),
        compiler_params=pltpu.CompilerParams(needs_layout_passes=False),
        scratch_types=[
            pltpu.VMEM((_CROWS, _W), jnp.int32),
            pltpu.VMEM((_CROWS, _W), jnp.int32),
            pltpu.VMEM((_CROWS, _W), jnp.int32),
            pltpu.VMEM((_CROWS, _W), jnp.int32),
            pltpu.VMEM((_HSIZE,), jnp.int32),
            pltpu.VMEM((_HSIZE,), jnp.int32),
            pltpu.VMEM((_LANES * 64,), jnp.int32),
            pltpu.VMEM((64,), jnp.int32),
            pltpu.SemaphoreType.DMA,
            pltpu.SemaphoreType.DMA,
            pltpu.SemaphoreType.DMA,
            pltpu.SemaphoreType.DMA,
        ],
    )


def _combine_body(parts_ref, out_ref):
    x = parts_ref[...].astype(jnp.float32)
    cp = jnp.zeros((_B, _C), jnp.float32)
    cl = jnp.zeros((_B, _C), jnp.float32)
    cb = jnp.zeros((_B, _C), jnp.float32)
    for q in range(4):
        o = q * 64
        cp = cp + x[:, o:o + _C]
        cl = cl + x[:, o + _C:o + 2 * _C]
        cb = cb + x[:, o + 2 * _C:o + 3 * _C]
    dice = (2.0 * cb) / (cp + cl + 1e-10)
    out_ref[...] = jnp.mean(dice, axis=0, keepdims=True)


def kernel(pred, label):
    pred_rows = pred.reshape(_ROWS, _W)
    label_rows = label.reshape(_ROWS, _W)
    parts = _sc_hist()(pred_rows, label_rows)
    parts2 = parts.reshape(_B, 4 * 64)
    out = pl.pallas_call(
        _combine_body,
        out_shape=jax.ShapeDtypeStruct((1, _C), jnp.float32),
    )(parts2)
    return out.reshape(_C)
